# baseline (speedup 1.0000x reference)
import functools

import jax
import jax.numpy as jnp
from jax import lax
from jax.experimental import pallas as pl
from jax.experimental.pallas import tpu as pltpu
from jax.experimental.pallas import tpu_sc as plsc

D = 128
B = 4096
L = 20
NSEQ = 6
R = NSEQ * B
NC = 2
NS = 16
NW = NC * NS
RW = R // NW
G = 6
NIDX = G * L
STEPS = RW // G
NDBLK = D // 16


NBUF = 8
OSTEP = NBUF * G


def _pool_body(table_hbm, ids_hbm, coef_hbm, out_hbm,
               ids_v, coef_v, rows0, rows1, rows2, rows3, rows4, rows5,
               rows6, rows7, out_v, g0, g1, g2, g3, g4, g5, g6, g7, osem):
    wid = lax.axis_index("s") * NC + lax.axis_index("c")
    base_row = wid * RW
    rbufs = (rows0, rows1, rows2, rows3, rows4, rows5, rows6, rows7)
    gsems = (g0, g1, g2, g3, g4, g5, g6, g7)
    ev_idx = jax.lax.iota(jnp.int32, 16) * 2

    pltpu.sync_copy(ids_hbm.at[pl.ds(base_row * L, RW * L)], ids_v)
    pltpu.sync_copy(coef_hbm.at[pl.ds(base_row, RW)], coef_v.at[pl.ds(0, RW)])

    def start_gather(s, rows_buf, sem):
        idx = ids_v.at[pl.ds(s * NIDX, NIDX)]
        pltpu.async_copy(table_hbm.at[idx], rows_buf, sem)

    def wait_gather(rows_buf, sem):
        pltpu.make_async_copy(table_hbm.at[pl.ds(0, NIDX)], rows_buf, sem).wait()

    def start_out(so):
        pltpu.async_copy(
            out_v, out_hbm.at[pl.ds((base_row + so * OSTEP) * D, OSTEP * D)],
            osem)

    def wait_out():
        pltpu.make_async_copy(
            out_v, out_hbm.at[pl.ds(0, OSTEP * D)], osem).wait()

    def compute(s, b, rows_buf):
        cvec = coef_v[pl.ds(s * G, 16)]
        for i in range(G):
            c = cvec[i]

            def dbody(dblk, _):
                sl = pl.ds(dblk * 16, 16)
                acc_a = [None] * 2
                acc_b = [None] * 2
                e0a = e0b = None
                for j in range(L):
                    v = rows_buf[i * L + j, sl]
                    va = plsc.bitcast(v << 16, jnp.float32)
                    vb = plsc.bitcast(v & jnp.int32(-65536), jnp.float32)
                    if j == 0:
                        e0a, e0b = va, vb
                    k = j % 2
                    acc_a[k] = va if acc_a[k] is None else acc_a[k] + va
                    acc_b[k] = vb if acc_b[k] is None else acc_b[k] + vb
                ra = (acc_a[0] + acc_a[1]) * (1.0 / L) + c * e0a
                rb = (acc_b[0] + acc_b[1]) * (1.0 / L) + c * e0b
                off = (b * G + i) * D + dblk * 32
                plsc.store_scatter(out_v, [ev_idx + off], ra)
                plsc.store_scatter(out_v, [ev_idx + (off + 1)], rb)
                return _

            lax.fori_loop(0, NDBLK // 2, dbody, 0)

    for b in range(NBUF):
        start_gather(b, rbufs[b], gsems[b])

    def body(so, carry):
        @pl.when(so >= 1)
        def _():
            wait_out()

        for b in range(NBUF):
            s = so * NBUF + b
            wait_gather(rbufs[b], gsems[b])
            compute(s, b, rbufs[b])

            @pl.when(s + NBUF < STEPS)
            def _():
                start_gather(s + NBUF, rbufs[b], gsems[b])

        start_out(so)
        return carry

    lax.fori_loop(0, STEPS // NBUF, body, 0)
    wait_out()


@functools.partial(
    pl.kernel,
    mesh=plsc.VectorSubcoreMesh(core_axis_name="c", subcore_axis_name="s"),
    compiler_params=pltpu.CompilerParams(needs_layout_passes=False,
                                         use_tc_tiling_on_sc=False),
    out_type=jax.ShapeDtypeStruct((R * D,), jnp.float32),
    scratch_types=[
        pltpu.VMEM((RW * L,), jnp.int32),
        pltpu.VMEM((RW + 16,), jnp.float32),
        pltpu.VMEM((NIDX, D // 2), jnp.int32),
        pltpu.VMEM((NIDX, D // 2), jnp.int32),
        pltpu.VMEM((NIDX, D // 2), jnp.int32),
        pltpu.VMEM((NIDX, D // 2), jnp.int32),
        pltpu.VMEM((NIDX, D // 2), jnp.int32),
        pltpu.VMEM((NIDX, D // 2), jnp.int32),
        pltpu.VMEM((NIDX, D // 2), jnp.int32),
        pltpu.VMEM((NIDX, D // 2), jnp.int32),
        pltpu.VMEM((OSTEP * D,), jnp.float32),
        pltpu.SemaphoreType.DMA,
        pltpu.SemaphoreType.DMA,
        pltpu.SemaphoreType.DMA,
        pltpu.SemaphoreType.DMA,
        pltpu.SemaphoreType.DMA,
        pltpu.SemaphoreType.DMA,
        pltpu.SemaphoreType.DMA,
        pltpu.SemaphoreType.DMA,
        pltpu.SemaphoreType.DMA,
    ],
)
def _pool(table_hbm, ids_hbm, coef_hbm, out_hbm, *rest):
    _pool_body(table_hbm, ids_hbm, coef_hbm, out_hbm, *rest)


def _fc_body(x_ref, w_ref, b_ref, o_ref):
    w = w_ref[...]
    acc = b_ref[0][None, :].astype(jnp.float32)
    for p in range(3):
        acc = acc + jax.lax.dot_general(
            x_ref[0, p], w[p * D:(p + 1) * D, :],
            (((1,), (0,)), ((), ())),
            preferred_element_type=jnp.float32,
            precision=jax.lax.Precision.HIGHEST,
        )
    o_ref[0] = acc


_RB = 512

_fc = pl.pallas_call(
    _fc_body,
    grid=(2, B // _RB),
    in_specs=[
        pl.BlockSpec((1, 3, _RB, D), lambda e, r: (e, 0, r, 0)),
        pl.BlockSpec((3 * D, D), lambda e, r: (0, 0)),
        pl.BlockSpec((1, D), lambda e, r: (0, 0)),
    ],
    out_specs=pl.BlockSpec((1, _RB, D), lambda e, r: (e, r, 0)),
    out_shape=jax.ShapeDtypeStruct((2, B, D), jnp.float32),
)


def kernel(table, fc_w, fc_b,
           evtq_s_ids, evtq_s_lengths, evtq_p_ids, evtq_p_lengths,
           evtq_o_ids, evtq_o_lengths,
           evtk_s_ids, evtk_s_lengths, evtk_p_ids, evtk_p_lengths,
           evtk_o_ids, evtk_o_lengths):
    ids_all = jnp.stack([evtq_s_ids, evtq_p_ids, evtq_o_ids,
                         evtk_s_ids, evtk_p_ids, evtk_o_ids])
    lens_all = jnp.stack([evtq_s_lengths, evtq_p_lengths, evtq_o_lengths,
                          evtk_s_lengths, evtk_p_lengths, evtk_o_lengths])
    pos = jnp.arange(L, dtype=lens_all.dtype)
    idsm = jnp.where(pos[None, None, :] < lens_all[:, :, None],
                     ids_all, ids_all[:, :, :1]).astype(jnp.int32)
    coef = (lens_all.astype(jnp.float32) - L) * (1.0 / L)

    table32 = jax.lax.bitcast_convert_type(
        table.astype(jnp.bfloat16).reshape(table.shape[0], D // 2, 2),
        jnp.int32)
    pooled = _pool(table32, idsm.reshape(-1), coef.reshape(-1))
    out2 = _fc(pooled.reshape(2, 3, B, D), fc_w, fc_b.reshape(1, D))
    return out2[0], out2[1]


# speedup vs baseline: 2.0135x; 2.0135x over previous
import functools

import jax
import jax.numpy as jnp
from jax import lax
from jax.experimental import pallas as pl
from jax.experimental.pallas import tpu as pltpu
from jax.experimental.pallas import tpu_sc as plsc

D = 128
B = 4096
L = 20
NSEQ = 6
R = NSEQ * B
NC = 2
NS = 16
NW = NC * NS
RW = R // NW
G = 6
NIDX = G * L
STEPS = RW // G
NDBLK = D // 16


NBUF = 8
OSTEP = NBUF * G


def _pool_body(table_hbm, ids_hbm, coef_hbm, out_hbm,
               ids_v, coef_v, rows0, rows1, rows2, rows3, rows4, rows5,
               rows6, rows7, out_v, g0, g1, g2, g3, g4, g5, g6, g7, osem):
    wid = lax.axis_index("s") * NC + lax.axis_index("c")
    base_row = wid * RW
    rbufs = (rows0, rows1, rows2, rows3, rows4, rows5, rows6, rows7)
    gsems = (g0, g1, g2, g3, g4, g5, g6, g7)

    pltpu.sync_copy(ids_hbm.at[pl.ds(base_row * L, RW * L)], ids_v)
    pltpu.sync_copy(coef_hbm.at[pl.ds(base_row, RW)], coef_v.at[pl.ds(0, RW)])

    def start_gather(s, rows_buf, sem):
        idx = ids_v.at[pl.ds(s * NIDX, NIDX)]
        pltpu.async_copy(table_hbm.at[idx], rows_buf, sem)

    def wait_gather(rows_buf, sem):
        pltpu.make_async_copy(table_hbm.at[pl.ds(0, NIDX)], rows_buf, sem).wait()

    def start_out(so):
        pltpu.async_copy(
            out_v, out_hbm.at[pl.ds((base_row + so * OSTEP) * D, OSTEP * D)],
            osem)

    def wait_out():
        pltpu.make_async_copy(
            out_v, out_hbm.at[pl.ds(0, OSTEP * D)], osem).wait()

    def compute(s, b, rows_buf):
        cvec = coef_v[pl.ds(s * G, 16)]
        for i in range(G):
            c = cvec[i]

            def dbody(dblk, _):
                sl = pl.ds(dblk * 16, 16)
                acc_a = [None] * 2
                acc_b = [None] * 2
                e0a = e0b = None
                for j in range(L):
                    v = rows_buf[i * L + j, sl]
                    va = plsc.bitcast(v << 16, jnp.float32)
                    vb = plsc.bitcast(v & jnp.int32(-65536), jnp.float32)
                    if j == 0:
                        e0a, e0b = va, vb
                    k = j % 2
                    acc_a[k] = va if acc_a[k] is None else acc_a[k] + va
                    acc_b[k] = vb if acc_b[k] is None else acc_b[k] + vb
                ra = (acc_a[0] + acc_a[1]) * (1.0 / L) + c * e0a
                rb = (acc_b[0] + acc_b[1]) * (1.0 / L) + c * e0b
                off = (b * G + i) * D + dblk * 16
                out_v[pl.ds(off, 16)] = ra
                out_v[pl.ds(off + 64, 16)] = rb
                return _

            lax.fori_loop(0, NDBLK // 2, dbody, 0)

    for b in range(NBUF):
        start_gather(b, rbufs[b], gsems[b])

    def body(so, carry):
        @pl.when(so >= 1)
        def _():
            wait_out()

        for b in range(NBUF):
            s = so * NBUF + b
            wait_gather(rbufs[b], gsems[b])
            compute(s, b, rbufs[b])

            @pl.when(s + NBUF < STEPS)
            def _():
                start_gather(s + NBUF, rbufs[b], gsems[b])

        start_out(so)
        return carry

    lax.fori_loop(0, STEPS // NBUF, body, 0)
    wait_out()


@functools.partial(
    pl.kernel,
    mesh=plsc.VectorSubcoreMesh(core_axis_name="c", subcore_axis_name="s"),
    compiler_params=pltpu.CompilerParams(needs_layout_passes=False,
                                         use_tc_tiling_on_sc=False),
    out_type=jax.ShapeDtypeStruct((R * D,), jnp.float32),
    scratch_types=[
        pltpu.VMEM((RW * L,), jnp.int32),
        pltpu.VMEM((RW + 16,), jnp.float32),
        pltpu.VMEM((NIDX, D // 2), jnp.int32),
        pltpu.VMEM((NIDX, D // 2), jnp.int32),
        pltpu.VMEM((NIDX, D // 2), jnp.int32),
        pltpu.VMEM((NIDX, D // 2), jnp.int32),
        pltpu.VMEM((NIDX, D // 2), jnp.int32),
        pltpu.VMEM((NIDX, D // 2), jnp.int32),
        pltpu.VMEM((NIDX, D // 2), jnp.int32),
        pltpu.VMEM((NIDX, D // 2), jnp.int32),
        pltpu.VMEM((OSTEP * D,), jnp.float32),
        pltpu.SemaphoreType.DMA,
        pltpu.SemaphoreType.DMA,
        pltpu.SemaphoreType.DMA,
        pltpu.SemaphoreType.DMA,
        pltpu.SemaphoreType.DMA,
        pltpu.SemaphoreType.DMA,
        pltpu.SemaphoreType.DMA,
        pltpu.SemaphoreType.DMA,
        pltpu.SemaphoreType.DMA,
    ],
)
def _pool(table_hbm, ids_hbm, coef_hbm, out_hbm, *rest):
    _pool_body(table_hbm, ids_hbm, coef_hbm, out_hbm, *rest)


def _fc_body(x_ref, w_ref, b_ref, o_ref):
    w = w_ref[...]
    acc = b_ref[0][None, :].astype(jnp.float32)
    for p in range(3):
        acc = acc + jax.lax.dot_general(
            x_ref[0, p], w[p * D:(p + 1) * D, :],
            (((1,), (0,)), ((), ())),
            preferred_element_type=jnp.float32,
            precision=jax.lax.Precision.HIGHEST,
        )
    o_ref[0] = acc


_RB = 512

_fc = pl.pallas_call(
    _fc_body,
    grid=(2, B // _RB),
    in_specs=[
        pl.BlockSpec((1, 3, _RB, D), lambda e, r: (e, 0, r, 0)),
        pl.BlockSpec((3 * D, D), lambda e, r: (0, 0)),
        pl.BlockSpec((1, D), lambda e, r: (0, 0)),
    ],
    out_specs=pl.BlockSpec((1, _RB, D), lambda e, r: (e, r, 0)),
    out_shape=jax.ShapeDtypeStruct((2, B, D), jnp.float32),
)


def kernel(table, fc_w, fc_b,
           evtq_s_ids, evtq_s_lengths, evtq_p_ids, evtq_p_lengths,
           evtq_o_ids, evtq_o_lengths,
           evtk_s_ids, evtk_s_lengths, evtk_p_ids, evtk_p_lengths,
           evtk_o_ids, evtk_o_lengths):
    ids_all = jnp.stack([evtq_s_ids, evtq_p_ids, evtq_o_ids,
                         evtk_s_ids, evtk_p_ids, evtk_o_ids])
    lens_all = jnp.stack([evtq_s_lengths, evtq_p_lengths, evtq_o_lengths,
                          evtk_s_lengths, evtk_p_lengths, evtk_o_lengths])
    pos = jnp.arange(L, dtype=lens_all.dtype)
    idsm = jnp.where(pos[None, None, :] < lens_all[:, :, None],
                     ids_all, ids_all[:, :, :1]).astype(jnp.int32)
    coef = (lens_all.astype(jnp.float32) - L) * (1.0 / L)

    u32 = jax.lax.bitcast_convert_type(
        table.astype(jnp.bfloat16), jnp.uint16).astype(jnp.uint32)
    table32 = jax.lax.bitcast_convert_type(
        u32[:, :D // 2] | (u32[:, D // 2:] << 16), jnp.int32)
    pooled = _pool(table32, idsm.reshape(-1), coef.reshape(-1))
    out2 = _fc(pooled.reshape(2, 3, B, D), fc_w, fc_b.reshape(1, D))
    return out2[0], out2[1]
